# baseline (speedup 1.0000x reference)
import jax
import jax.numpy as jnp
from jax.experimental import pallas as pl
from jax.experimental.pallas import tpu as pltpu

_BLK = 4096
_C_PAD = 128
_C_RED = 104
_ROWS = 8


def _half(emb, w, kap, bias):
    norm2 = jnp.sum(emb * emb, axis=1, keepdims=True)
    inv = jnp.float32(1.0) / jnp.maximum(jnp.sqrt(norm2),
                                         jnp.float32(1e-12))
    emb_n = (emb * inv).astype(jnp.bfloat16)
    dot = jax.lax.dot_general(
        w, emb_n, (((1,), (1,)), ((), ())),
        preferred_element_type=jnp.float32)
    logits = dot[:_C_RED] * kap + bias
    m = jnp.max(logits, axis=0, keepdims=True)
    s = jnp.sum(jnp.exp(logits - m), axis=0, keepdims=True)
    row = jax.lax.broadcasted_iota(jnp.int32, (_C_RED, _BLK), 0)
    idx = jnp.min(jnp.where(logits == m, row, jnp.int32(_C_RED)),
                  axis=0, keepdims=True)
    return -(m + jnp.log(s)), idx


def _fused_kernel(a_ref, b_ref, w_ref, kap_ref, bias_ref,
                  energy_ref, pred_ref):
    w = w_ref[...]
    kap = kap_ref[:, :1]
    bias = bias_ref[:, :1]
    e0, i0 = _half(a_ref[...], w, kap, bias)
    e1, i1 = _half(b_ref[...], w, kap, bias)
    r = pl.program_id(0) % _ROWS
    energy_ref[pl.ds(2 * r, 1), :] = e0
    energy_ref[pl.ds(2 * r + 1, 1), :] = e1
    pred_ref[pl.ds(2 * r, 1), :] = i0
    pred_ref[pl.ds(2 * r + 1, 1), :] = i1


def kernel(embedding_3d, mus, kappas, classes):
    B, F, D, H, W = embedding_3d.shape
    N = B * D * H * W
    C = mus.shape[0]
    emb_v = embedding_3d.transpose(0, 2, 3, 4, 1).reshape(N, F)
    w = jnp.zeros((_C_PAD, F), jnp.bfloat16).at[:C].set(
        mus.astype(jnp.bfloat16))
    kap = jnp.zeros((_C_RED, 128), jnp.float32).at[:C].set(
        kappas[:, None])
    bias = jnp.full((_C_RED, 128), -1e30, jnp.float32).at[:C].set(0.0)

    grid = (N // (2 * _BLK),)
    energy, pred = pl.pallas_call(
        _fused_kernel,
        grid=grid,
        compiler_params=pltpu.CompilerParams(
            dimension_semantics=("parallel",)),
        in_specs=[
            pl.BlockSpec((_BLK, F), lambda i: (2 * i, 0)),
            pl.BlockSpec((_BLK, F), lambda i: (2 * i + 1, 0)),
            pl.BlockSpec((_C_PAD, F), lambda i: (0, 0)),
            pl.BlockSpec((_C_RED, 128), lambda i: (0, 0)),
            pl.BlockSpec((_C_RED, 128), lambda i: (0, 0)),
        ],
        out_specs=[
            pl.BlockSpec((2 * _ROWS, _BLK), lambda i: (i // _ROWS, 0)),
            pl.BlockSpec((2 * _ROWS, _BLK), lambda i: (i // _ROWS, 0)),
        ],
        out_shape=[
            jax.ShapeDtypeStruct((N // _BLK, _BLK), jnp.float32),
            jax.ShapeDtypeStruct((N // _BLK, _BLK), jnp.int32),
        ],
    )(emb_v, emb_v, w, kap, bias)
    return (energy.reshape(B, D, H, W),
            pred.reshape(B, D, H, W))


# speedup vs baseline: 1.0163x; 1.0163x over previous
import jax
import jax.numpy as jnp
from jax.experimental import pallas as pl
from jax.experimental.pallas import tpu as pltpu

_BLK = 2048
_C_PAD = 128
_C_RED = 104
_ROWS = 8


def _half(emb, w, kap, bias):
    norm2 = jnp.sum(emb * emb, axis=1, keepdims=True)
    inv = jnp.float32(1.0) / jnp.maximum(jnp.sqrt(norm2),
                                         jnp.float32(1e-12))
    emb_n = (emb * inv).astype(jnp.bfloat16)
    dot = jax.lax.dot_general(
        w, emb_n, (((1,), (1,)), ((), ())),
        preferred_element_type=jnp.float32)
    logits = dot[:_C_RED] * kap + bias
    m = jnp.max(logits, axis=0, keepdims=True)
    s = jnp.sum(jnp.exp(logits - m), axis=0, keepdims=True)
    row = jax.lax.broadcasted_iota(jnp.int32, (_C_RED, _BLK), 0)
    idx = jnp.min(jnp.where(logits == m, row, jnp.int32(_C_RED)),
                  axis=0, keepdims=True)
    return -(m + jnp.log(s)), idx


def _fused_kernel(a_ref, b_ref, w_ref, kap_ref, bias_ref,
                  energy_ref, pred_ref):
    w = w_ref[...]
    kap = kap_ref[:, :1]
    bias = bias_ref[:, :1]
    e0, i0 = _half(a_ref[...], w, kap, bias)
    e1, i1 = _half(b_ref[...], w, kap, bias)
    r = pl.program_id(0) % _ROWS
    energy_ref[pl.ds(2 * r, 1), :] = e0
    energy_ref[pl.ds(2 * r + 1, 1), :] = e1
    pred_ref[pl.ds(2 * r, 1), :] = i0
    pred_ref[pl.ds(2 * r + 1, 1), :] = i1


def kernel(embedding_3d, mus, kappas, classes):
    B, F, D, H, W = embedding_3d.shape
    N = B * D * H * W
    C = mus.shape[0]
    emb_v = embedding_3d.transpose(0, 2, 3, 4, 1).reshape(N, F)
    w = jnp.zeros((_C_PAD, F), jnp.bfloat16).at[:C].set(
        mus.astype(jnp.bfloat16))
    kap = jnp.zeros((_C_RED, 128), jnp.float32).at[:C].set(
        kappas[:, None])
    bias = jnp.full((_C_RED, 128), -1e30, jnp.float32).at[:C].set(0.0)

    grid = (N // (2 * _BLK),)
    energy, pred = pl.pallas_call(
        _fused_kernel,
        grid=grid,
        compiler_params=pltpu.CompilerParams(
            dimension_semantics=("parallel",)),
        in_specs=[
            pl.BlockSpec((_BLK, F), lambda i: (2 * i, 0)),
            pl.BlockSpec((_BLK, F), lambda i: (2 * i + 1, 0)),
            pl.BlockSpec((_C_PAD, F), lambda i: (0, 0)),
            pl.BlockSpec((_C_RED, 128), lambda i: (0, 0)),
            pl.BlockSpec((_C_RED, 128), lambda i: (0, 0)),
        ],
        out_specs=[
            pl.BlockSpec((2 * _ROWS, _BLK), lambda i: (i // _ROWS, 0)),
            pl.BlockSpec((2 * _ROWS, _BLK), lambda i: (i // _ROWS, 0)),
        ],
        out_shape=[
            jax.ShapeDtypeStruct((N // _BLK, _BLK), jnp.float32),
            jax.ShapeDtypeStruct((N // _BLK, _BLK), jnp.int32),
        ],
    )(emb_v, emb_v, w, kap, bias)
    return (energy.reshape(B, D, H, W),
            pred.reshape(B, D, H, W))
